# baseline (speedup 1.0000x reference)
import functools

import jax
import jax.numpy as jnp
from jax import lax
from jax.experimental import pallas as pl
from jax.experimental.pallas import tpu as pltpu
from jax.experimental.pallas import tpu_sc as plsc

N_V, N_E, NNZ = 10000, 5000, 320000
D_IN, N_CLS = 128, 64
VP, EP = 10240, 5120
NC, NS = 2, 16
NW = NC * NS
CH = 8000
NCH = NNZ // CH
PPT = NNZ // NW

_SC_PARAMS = pltpu.CompilerParams(needs_layout_passes=False)
_MESH = plsc.VectorSubcoreMesh(core_axis_name="c", subcore_axis_name="s")


def _wid():
    return lax.axis_index("s") * NC + lax.axis_index("c")


def _zero_fill(ref, nvecs):
    z16 = jnp.zeros((16,), jnp.float32)

    @plsc.parallel_loop(0, nvecs, unroll=8)
    def _(i):
        ref[pl.ds(i * 16, 16)] = z16


@functools.partial(
    pl.kernel, mesh=_MESH, compiler_params=_SC_PARAMS,
    out_type=(jax.ShapeDtypeStruct((NW * VP,), jnp.float32),
              jax.ShapeDtypeStruct((NW * EP,), jnp.float32)),
    scratch_types=[pltpu.VMEM((VP,), jnp.float32),
                   pltpu.VMEM((EP,), jnp.float32),
                   pltpu.VMEM((PPT,), jnp.int32),
                   pltpu.VMEM((PPT,), jnp.int32)],
)
def _sc_hist(v_hbm, e_hbm, dvp_hbm, esp_hbm, dvh, esh, vb, eb):
    wid = _wid()
    _zero_fill(dvh, VP // 16)
    _zero_fill(esh, EP // 16)
    base = wid * PPT
    pltpu.sync_copy(v_hbm.at[pl.ds(base, PPT)], vb)
    pltpu.sync_copy(e_hbm.at[pl.ds(base, PPT)], eb)
    ones = jnp.full((16,), 1.0, jnp.float32)

    @plsc.parallel_loop(0, PPT // 16, unroll=4)
    def _(g):
        vv = vb[pl.ds(g * 16, 16)]
        ve = eb[pl.ds(g * 16, 16)]
        plsc.addupdate_scatter(dvh, [vv], ones)
        plsc.addupdate_scatter(esh, [ve], ones)
    pltpu.sync_copy(dvh, dvp_hbm.at[pl.ds(wid * VP, VP)])
    pltpu.sync_copy(esh, esp_hbm.at[pl.ds(wid * EP, EP)])


@functools.partial(
    pl.kernel, mesh=_MESH, compiler_params=_SC_PARAMS,
    out_type=jax.ShapeDtypeStruct((NW * EP,), jnp.float32),
    scratch_types=[pltpu.VMEM((VP,), jnp.float32),
                   pltpu.VMEM((EP,), jnp.float32),
                   pltpu.VMEM((PPT,), jnp.int32),
                   pltpu.VMEM((PPT,), jnp.int32)],
)
def _sc_te(v_hbm, e_hbm, dv_hbm, tep_hbm, dvb, teh, vb, eb):
    wid = _wid()
    _zero_fill(teh, EP // 16)
    pltpu.sync_copy(dv_hbm, dvb)
    base = wid * PPT
    pltpu.sync_copy(v_hbm.at[pl.ds(base, PPT)], vb)
    pltpu.sync_copy(e_hbm.at[pl.ds(base, PPT)], eb)

    @plsc.parallel_loop(0, PPT // 16, unroll=4)
    def _(g):
        vv = vb[pl.ds(g * 16, 16)]
        ve = eb[pl.ds(g * 16, 16)]
        vals = plsc.load_gather(dvb, [vv])
        plsc.addupdate_scatter(teh, [ve], vals)
    pltpu.sync_copy(teh, tep_hbm.at[pl.ds(wid * EP, EP)])


def _make_sc_layer(dl, relu):
    fpt = dl // NW
    gpc = CH // 16

    @functools.partial(
        pl.kernel, mesh=_MESH, compiler_params=_SC_PARAMS,
        out_type=jax.ShapeDtypeStruct((dl * VP,), jnp.float32),
        scratch_types=[pltpu.VMEM((fpt * VP,), jnp.float32),
                       pltpu.VMEM((fpt * EP,), jnp.float32),
                       pltpu.VMEM((EP,), jnp.float32),
                       pltpu.VMEM((VP,), jnp.float32),
                       pltpu.VMEM((CH,), jnp.int32),
                       pltpu.VMEM((CH,), jnp.int32),
                       pltpu.VMEM((CH,), jnp.int32),
                       pltpu.VMEM((CH,), jnp.int32),
                       pltpu.SemaphoreType.DMA,
                       pltpu.SemaphoreType.DMA],
    )
    def sc_layer(xp_hbm, v_hbm, e_hbm, vs_hbm, es_hbm, s_hbm, ivd_hbm,
                 out_hbm, xp, ze, sbuf, ivdbuf, vb0, vb1, eb0, eb1,
                 sem0, sem1):
        wid = _wid()
        base = wid * fpt * VP
        pltpu.sync_copy(xp_hbm.at[pl.ds(base, fpt * VP)], xp)
        pltpu.sync_copy(s_hbm, sbuf)
        pltpu.sync_copy(ivd_hbm, ivdbuf)

        def pass_a(vb, eb):
            @plsc.parallel_loop(0, gpc, unroll=8)
            def _(g):
                vv = vb[pl.ds(g * 16, 16)]
                ve = eb[pl.ds(g * 16, 16)]
                for f in range(fpt):
                    vals = plsc.load_gather(xp, [vv + (f * VP)])
                    plsc.addupdate_scatter(ze, [ve + (f * EP)], vals)

        def pass_b(vb, eb):
            @plsc.parallel_loop(0, gpc, unroll=8)
            def _(g):
                vv = vb[pl.ds(g * 16, 16)]
                ve = eb[pl.ds(g * 16, 16)]
                for f in range(fpt):
                    vals = plsc.load_gather(ze, [ve + (f * EP)])
                    plsc.addupdate_scatter(xp, [vv + (f * VP)], vals)

        def run_pass(chunk_fn, vsrc, esrc):
            bufs = ((vb0, eb0, sem0), (vb1, eb1, sem1))
            for k, (vb, eb, sem) in enumerate(bufs):
                pltpu.async_copy(vsrc.at[pl.ds(k * CH, CH)], vb, sem)
                pltpu.async_copy(esrc.at[pl.ds(k * CH, CH)], eb, sem)

            def outer(cc, _):
                c0 = cc * 2
                for k, (vb, eb, sem) in enumerate(bufs):
                    c = c0 + k
                    pltpu.make_async_copy(vsrc.at[pl.ds(0, CH)], vb, sem).wait()
                    pltpu.make_async_copy(esrc.at[pl.ds(0, CH)], eb, sem).wait()
                    chunk_fn(vb, eb)

                    @pl.when(c + 2 < NCH)
                    def _():
                        pltpu.async_copy(vsrc.at[pl.ds((c + 2) * CH, CH)], vb, sem)
                        pltpu.async_copy(esrc.at[pl.ds((c + 2) * CH, CH)], eb, sem)

                return 0

            lax.fori_loop(0, NCH // 2, outer, 0)

        _zero_fill(ze, fpt * EP // 16)
        run_pass(pass_a, v_hbm, e_hbm)

        for f in range(fpt):
            @plsc.parallel_loop(0, EP // 16, unroll=4)
            def _(i, f=f):
                off = f * EP + i * 16
                ze[pl.ds(off, 16)] = ze[pl.ds(off, 16)] * sbuf[pl.ds(i * 16, 16)]

        _zero_fill(xp, fpt * VP // 16)
        run_pass(pass_b, vs_hbm, es_hbm)

        for f in range(fpt):
            @plsc.parallel_loop(0, VP // 16, unroll=4)
            def _(i, f=f):
                off = f * VP + i * 16
                av = xp[pl.ds(off, 16)] * ivdbuf[pl.ds(i * 16, 16)]
                if relu:
                    av = jnp.maximum(av, 0.0)
                xp[pl.ds(off, 16)] = av
        pltpu.sync_copy(xp, out_hbm.at[pl.ds(base, fpt * VP)])

    return sc_layer


_sc_layer_hidden = _make_sc_layer(D_IN, relu=True)
_sc_layer_final = _make_sc_layer(N_CLS, relu=False)


def _mm_body(wt_ref, x_ref, b_ref, o_ref):
    o_ref[...] = jnp.dot(wt_ref[...], x_ref[...],
                         preferred_element_type=jnp.float32) + b_ref[...]


def _mm(wt, x2d, b2d):
    do, di = wt.shape
    bv = 2560
    return pl.pallas_call(
        _mm_body,
        grid=(VP // bv,),
        in_specs=[pl.BlockSpec((do, di), lambda j: (0, 0)),
                  pl.BlockSpec((di, bv), lambda j: (0, j)),
                  pl.BlockSpec((do, 1), lambda j: (0, 0))],
        out_specs=pl.BlockSpec((do, bv), lambda j: (0, j)),
        out_shape=jax.ShapeDtypeStruct((do, VP), jnp.float32),
    )(wt, x2d, b2d)


def _comb1_body(dvp_ref, esp_ref, dv_ref, ivd_ref, ies_ref):
    dv = jnp.maximum(jnp.sum(dvp_ref[...], axis=0, keepdims=True), 1.0)
    dv_ref[...] = dv
    ivd_ref[...] = lax.rsqrt(dv)
    es = jnp.maximum(jnp.sum(esp_ref[...], axis=0, keepdims=True), 1.0)
    ies_ref[...] = 1.0 / es


def _comb1(dvp, esp):
    return pl.pallas_call(
        _comb1_body,
        out_shape=(jax.ShapeDtypeStruct((1, VP), jnp.float32),
                   jax.ShapeDtypeStruct((1, VP), jnp.float32),
                   jax.ShapeDtypeStruct((1, EP), jnp.float32)),
    )(dvp, esp)


def _comb2_body(tep_ref, ies_ref, s_ref):
    te = jnp.sum(tep_ref[...], axis=0, keepdims=True)
    ies = ies_ref[...]
    tilde = jnp.maximum(te * ies, 1.0)
    s_ref[...] = lax.rsqrt(tilde) * ies


def _comb2(tep, ies):
    return pl.pallas_call(
        _comb2_body,
        out_shape=jax.ShapeDtypeStruct((1, EP), jnp.float32),
    )(tep, ies)


def kernel(X, v_idx, e_idx, W0, b0, W1, b1, W2, b2, W3, b3):
    gpc = CH // 16

    def _stride(a):
        return a.reshape(NCH, 16, gpc).transpose(0, 2, 1).reshape(-1)

    v_so = v_idx.astype(jnp.int32)
    e_so = e_idx.astype(jnp.int32)
    v32 = _stride(v_so)
    e32 = _stride(e_so)

    dvp, esp = _sc_hist(v32, e32)
    dv2, ivd2, ies2 = _comb1(dvp.reshape(NW, VP), esp.reshape(NW, EP))
    tep = _sc_te(v32, e32, dv2.reshape(VP))
    s2 = _comb2(tep.reshape(NW, EP), ies2)
    s1 = s2.reshape(EP)
    ivd1 = ivd2.reshape(VP)

    h = jnp.zeros((D_IN, VP), jnp.float32).at[:, :N_V].set(X.T)
    layers = [(W0, b0), (W1, b1), (W2, b2), (W3, b3)]
    for l, (W, b) in enumerate(layers):
        xp = _mm(W.T, h, b[:, None])
        do = W.shape[1]
        sc = _sc_layer_final if l == 3 else _sc_layer_hidden
        flat = sc(xp.reshape(do * VP), v32, e32, v_so, e_so, s1, ivd1)
        h = flat.reshape(do, VP)
    return h[:, :N_V].T


# speedup vs baseline: 1.0086x; 1.0086x over previous
import functools

import jax
import jax.numpy as jnp
from jax import lax
from jax.experimental import pallas as pl
from jax.experimental.pallas import tpu as pltpu
from jax.experimental.pallas import tpu_sc as plsc

N_V, N_E, NNZ = 10000, 5000, 320000
D_IN, N_CLS = 128, 64
VP, EP = 10240, 5120
NC, NS = 2, 16
NW = NC * NS
CH = 8000
NCH = NNZ // CH
PPT = NNZ // NW

_SC_PARAMS = pltpu.CompilerParams(needs_layout_passes=False)
_MESH = plsc.VectorSubcoreMesh(core_axis_name="c", subcore_axis_name="s")


def _wid():
    return lax.axis_index("s") * NC + lax.axis_index("c")


def _zero_fill(ref, nvecs):
    z16 = jnp.zeros((16,), jnp.float32)

    @plsc.parallel_loop(0, nvecs, unroll=8)
    def _(i):
        ref[pl.ds(i * 16, 16)] = z16


@functools.partial(
    pl.kernel, mesh=_MESH, compiler_params=_SC_PARAMS,
    out_type=(jax.ShapeDtypeStruct((NW * VP,), jnp.float32),
              jax.ShapeDtypeStruct((NW * EP,), jnp.float32)),
    scratch_types=[pltpu.VMEM((VP,), jnp.float32),
                   pltpu.VMEM((EP,), jnp.float32),
                   pltpu.VMEM((PPT,), jnp.int32),
                   pltpu.VMEM((PPT,), jnp.int32)],
)
def _sc_hist(v_hbm, e_hbm, dvp_hbm, esp_hbm, dvh, esh, vb, eb):
    wid = _wid()
    _zero_fill(dvh, VP // 16)
    _zero_fill(esh, EP // 16)
    base = wid * PPT
    pltpu.sync_copy(v_hbm.at[pl.ds(base, PPT)], vb)
    pltpu.sync_copy(e_hbm.at[pl.ds(base, PPT)], eb)
    ones = jnp.full((16,), 1.0, jnp.float32)

    @plsc.parallel_loop(0, PPT // 16, unroll=4)
    def _(g):
        vv = vb[pl.ds(g * 16, 16)]
        ve = eb[pl.ds(g * 16, 16)]
        plsc.addupdate_scatter(dvh, [vv], ones)
        plsc.addupdate_scatter(esh, [ve], ones)
    pltpu.sync_copy(dvh, dvp_hbm.at[pl.ds(wid * VP, VP)])
    pltpu.sync_copy(esh, esp_hbm.at[pl.ds(wid * EP, EP)])


@functools.partial(
    pl.kernel, mesh=_MESH, compiler_params=_SC_PARAMS,
    out_type=jax.ShapeDtypeStruct((NW * EP,), jnp.float32),
    scratch_types=[pltpu.VMEM((VP,), jnp.float32),
                   pltpu.VMEM((EP,), jnp.float32),
                   pltpu.VMEM((PPT,), jnp.int32),
                   pltpu.VMEM((PPT,), jnp.int32)],
)
def _sc_te(v_hbm, e_hbm, dv_hbm, tep_hbm, dvb, teh, vb, eb):
    wid = _wid()
    _zero_fill(teh, EP // 16)
    pltpu.sync_copy(dv_hbm, dvb)
    base = wid * PPT
    pltpu.sync_copy(v_hbm.at[pl.ds(base, PPT)], vb)
    pltpu.sync_copy(e_hbm.at[pl.ds(base, PPT)], eb)

    @plsc.parallel_loop(0, PPT // 16, unroll=4)
    def _(g):
        vv = vb[pl.ds(g * 16, 16)]
        ve = eb[pl.ds(g * 16, 16)]
        vals = plsc.load_gather(dvb, [vv])
        plsc.addupdate_scatter(teh, [ve], vals)
    pltpu.sync_copy(teh, tep_hbm.at[pl.ds(wid * EP, EP)])


def _make_sc_layer(dl, relu):
    fpt = dl // NW
    gpc = CH // 16

    @functools.partial(
        pl.kernel, mesh=_MESH, compiler_params=_SC_PARAMS,
        out_type=jax.ShapeDtypeStruct((dl * VP,), jnp.float32),
        scratch_types=[pltpu.VMEM((fpt * VP,), jnp.float32),
                       pltpu.VMEM((fpt * EP,), jnp.float32),
                       pltpu.VMEM((EP,), jnp.float32),
                       pltpu.VMEM((VP,), jnp.float32),
                       pltpu.VMEM((CH,), jnp.int32),
                       pltpu.VMEM((CH,), jnp.int32),
                       pltpu.VMEM((CH,), jnp.int32),
                       pltpu.VMEM((CH,), jnp.int32),
                       pltpu.SemaphoreType.DMA,
                       pltpu.SemaphoreType.DMA],
    )
    def sc_layer(xp_hbm, v_hbm, e_hbm, vs_hbm, es_hbm, s_hbm, ivd_hbm,
                 out_hbm, xp, ze, sbuf, ivdbuf, vb0, vb1, eb0, eb1,
                 sem0, sem1):
        wid = _wid()
        base = wid * fpt * VP
        pltpu.sync_copy(xp_hbm.at[pl.ds(base, fpt * VP)], xp)
        pltpu.sync_copy(s_hbm, sbuf)
        pltpu.sync_copy(ivd_hbm, ivdbuf)

        def pass_a(vb, eb):
            @plsc.parallel_loop(0, gpc, unroll=4)
            def _(g):
                vv = vb[pl.ds(g * 16, 16)]
                ve = eb[pl.ds(g * 16, 16)]
                for f in range(fpt):
                    vals = plsc.load_gather(xp, [vv + (f * VP)])
                    plsc.addupdate_scatter(ze, [ve + (f * EP)], vals)

        def pass_b(vb, eb):
            @plsc.parallel_loop(0, gpc, unroll=4)
            def _(g):
                vv = vb[pl.ds(g * 16, 16)]
                ve = eb[pl.ds(g * 16, 16)]
                for f in range(fpt):
                    vals = plsc.load_gather(ze, [ve + (f * EP)])
                    plsc.addupdate_scatter(xp, [vv + (f * VP)], vals)

        def run_pass(chunk_fn, vsrc, esrc):
            bufs = ((vb0, eb0, sem0), (vb1, eb1, sem1))
            handles = {}

            def start(c):
                vb, eb, sem = bufs[c & 1]
                h1 = pltpu.async_copy(vsrc.at[pl.ds(c * CH, CH)], vb, sem)
                h2 = pltpu.async_copy(esrc.at[pl.ds(c * CH, CH)], eb, sem)
                handles[c] = (h1, h2)

            start(0)
            for c in range(NCH):
                h1, h2 = handles.pop(c)
                h1.wait()
                h2.wait()
                if c + 1 < NCH:
                    start(c + 1)
                vb, eb, _ = bufs[c & 1]
                chunk_fn(vb, eb)

        _zero_fill(ze, fpt * EP // 16)
        run_pass(pass_a, v_hbm, e_hbm)

        for f in range(fpt):
            @plsc.parallel_loop(0, EP // 16, unroll=4)
            def _(i, f=f):
                off = f * EP + i * 16
                ze[pl.ds(off, 16)] = ze[pl.ds(off, 16)] * sbuf[pl.ds(i * 16, 16)]

        _zero_fill(xp, fpt * VP // 16)
        run_pass(pass_b, vs_hbm, es_hbm)

        for f in range(fpt):
            @plsc.parallel_loop(0, VP // 16, unroll=4)
            def _(i, f=f):
                off = f * VP + i * 16
                av = xp[pl.ds(off, 16)] * ivdbuf[pl.ds(i * 16, 16)]
                if relu:
                    av = jnp.maximum(av, 0.0)
                xp[pl.ds(off, 16)] = av
        pltpu.sync_copy(xp, out_hbm.at[pl.ds(base, fpt * VP)])

    return sc_layer


_sc_layer_hidden = _make_sc_layer(D_IN, relu=True)
_sc_layer_final = _make_sc_layer(N_CLS, relu=False)


def _mm_body(wt_ref, x_ref, b_ref, o_ref):
    o_ref[...] = jnp.dot(wt_ref[...], x_ref[...],
                         preferred_element_type=jnp.float32) + b_ref[...]


def _mm(wt, x2d, b2d):
    do, di = wt.shape
    bv = 2560
    return pl.pallas_call(
        _mm_body,
        grid=(VP // bv,),
        in_specs=[pl.BlockSpec((do, di), lambda j: (0, 0)),
                  pl.BlockSpec((di, bv), lambda j: (0, j)),
                  pl.BlockSpec((do, 1), lambda j: (0, 0))],
        out_specs=pl.BlockSpec((do, bv), lambda j: (0, j)),
        out_shape=jax.ShapeDtypeStruct((do, VP), jnp.float32),
    )(wt, x2d, b2d)


def _comb1_body(dvp_ref, esp_ref, dv_ref, ivd_ref, ies_ref):
    dv = jnp.maximum(jnp.sum(dvp_ref[...], axis=0, keepdims=True), 1.0)
    dv_ref[...] = dv
    ivd_ref[...] = lax.rsqrt(dv)
    es = jnp.maximum(jnp.sum(esp_ref[...], axis=0, keepdims=True), 1.0)
    ies_ref[...] = 1.0 / es


def _comb1(dvp, esp):
    return pl.pallas_call(
        _comb1_body,
        out_shape=(jax.ShapeDtypeStruct((1, VP), jnp.float32),
                   jax.ShapeDtypeStruct((1, VP), jnp.float32),
                   jax.ShapeDtypeStruct((1, EP), jnp.float32)),
    )(dvp, esp)


def _comb2_body(tep_ref, ies_ref, s_ref):
    te = jnp.sum(tep_ref[...], axis=0, keepdims=True)
    ies = ies_ref[...]
    tilde = jnp.maximum(te * ies, 1.0)
    s_ref[...] = lax.rsqrt(tilde) * ies


def _comb2(tep, ies):
    return pl.pallas_call(
        _comb2_body,
        out_shape=jax.ShapeDtypeStruct((1, EP), jnp.float32),
    )(tep, ies)


def kernel(X, v_idx, e_idx, W0, b0, W1, b1, W2, b2, W3, b3):
    gpc = CH // 16

    def _stride(a):
        return a.reshape(NCH, 16, gpc).transpose(0, 2, 1).reshape(-1)

    v_so = v_idx.astype(jnp.int32)
    e_so = e_idx.astype(jnp.int32)
    v32 = _stride(v_so)
    e32 = _stride(e_so)

    dvp, esp = _sc_hist(v32, e32)
    dv2, ivd2, ies2 = _comb1(dvp.reshape(NW, VP), esp.reshape(NW, EP))
    tep = _sc_te(v32, e32, dv2.reshape(VP))
    s2 = _comb2(tep.reshape(NW, EP), ies2)
    s1 = s2.reshape(EP)
    ivd1 = ivd2.reshape(VP)

    h = jnp.zeros((D_IN, VP), jnp.float32).at[:, :N_V].set(X.T)
    layers = [(W0, b0), (W1, b1), (W2, b2), (W3, b3)]
    for l, (W, b) in enumerate(layers):
        xp = _mm(W.T, h, b[:, None])
        do = W.shape[1]
        sc = _sc_layer_final if l == 3 else _sc_layer_hidden
        flat = sc(xp.reshape(do * VP), v32, e32, v_so, e_so, s1, ivd1)
        h = flat.reshape(do, VP)
    return h[:, :N_V].T


# speedup vs baseline: 1.0179x; 1.0092x over previous
import functools

import jax
import jax.numpy as jnp
from jax import lax
from jax.experimental import pallas as pl
from jax.experimental.pallas import tpu as pltpu
from jax.experimental.pallas import tpu_sc as plsc

N_V, N_E, NNZ = 10000, 5000, 320000
D_IN, N_CLS = 128, 64
VP, EP = 10240, 5120
NC, NS = 2, 16
NW = NC * NS
CH = 8000
NCH = NNZ // CH
PPT = NNZ // NW

_SC_PARAMS = pltpu.CompilerParams(needs_layout_passes=False)
_MESH = plsc.VectorSubcoreMesh(core_axis_name="c", subcore_axis_name="s")


def _wid():
    return lax.axis_index("s") * NC + lax.axis_index("c")


def _zero_fill(ref, nvecs):
    z16 = jnp.zeros((16,), jnp.float32)

    @plsc.parallel_loop(0, nvecs, unroll=8)
    def _(i):
        ref[pl.ds(i * 16, 16)] = z16


@functools.partial(
    pl.kernel, mesh=_MESH, compiler_params=_SC_PARAMS,
    out_type=(jax.ShapeDtypeStruct((NW * VP,), jnp.float32),
              jax.ShapeDtypeStruct((NW * EP,), jnp.float32)),
    scratch_types=[pltpu.VMEM((VP,), jnp.float32),
                   pltpu.VMEM((EP,), jnp.float32),
                   pltpu.VMEM((PPT,), jnp.int32),
                   pltpu.VMEM((PPT,), jnp.int32)],
)
def _sc_hist(v_hbm, e_hbm, dvp_hbm, esp_hbm, dvh, esh, vb, eb):
    wid = _wid()
    _zero_fill(dvh, VP // 16)
    _zero_fill(esh, EP // 16)
    base = wid * PPT
    pltpu.sync_copy(v_hbm.at[pl.ds(base, PPT)], vb)
    pltpu.sync_copy(e_hbm.at[pl.ds(base, PPT)], eb)
    ones = jnp.full((16,), 1.0, jnp.float32)

    @plsc.parallel_loop(0, PPT // 16, unroll=4)
    def _(g):
        vv = vb[pl.ds(g * 16, 16)]
        ve = eb[pl.ds(g * 16, 16)]
        plsc.addupdate_scatter(dvh, [vv], ones)
        plsc.addupdate_scatter(esh, [ve], ones)
    pltpu.sync_copy(dvh, dvp_hbm.at[pl.ds(wid * VP, VP)])
    pltpu.sync_copy(esh, esp_hbm.at[pl.ds(wid * EP, EP)])


@functools.partial(
    pl.kernel, mesh=_MESH, compiler_params=_SC_PARAMS,
    out_type=jax.ShapeDtypeStruct((NW * EP,), jnp.float32),
    scratch_types=[pltpu.VMEM((VP,), jnp.float32),
                   pltpu.VMEM((EP,), jnp.float32),
                   pltpu.VMEM((PPT,), jnp.int32),
                   pltpu.VMEM((PPT,), jnp.int32)],
)
def _sc_te(v_hbm, e_hbm, dv_hbm, tep_hbm, dvb, teh, vb, eb):
    wid = _wid()
    _zero_fill(teh, EP // 16)
    pltpu.sync_copy(dv_hbm, dvb)
    base = wid * PPT
    pltpu.sync_copy(v_hbm.at[pl.ds(base, PPT)], vb)
    pltpu.sync_copy(e_hbm.at[pl.ds(base, PPT)], eb)

    @plsc.parallel_loop(0, PPT // 16, unroll=4)
    def _(g):
        vv = vb[pl.ds(g * 16, 16)]
        ve = eb[pl.ds(g * 16, 16)]
        vals = plsc.load_gather(dvb, [vv])
        plsc.addupdate_scatter(teh, [ve], vals)
    pltpu.sync_copy(teh, tep_hbm.at[pl.ds(wid * EP, EP)])


def _make_sc_layer(dl, apply_ivd):
    fpt = dl // NW
    gpc = CH // 16

    @functools.partial(
        pl.kernel, mesh=_MESH, compiler_params=_SC_PARAMS,
        out_type=jax.ShapeDtypeStruct((dl * VP,), jnp.float32),
        scratch_types=[pltpu.VMEM((fpt * VP,), jnp.float32),
                       pltpu.VMEM((fpt * EP,), jnp.float32),
                       pltpu.VMEM((EP,), jnp.float32),
                       pltpu.VMEM((VP,), jnp.float32),
                       pltpu.VMEM((CH,), jnp.int32),
                       pltpu.VMEM((CH,), jnp.int32),
                       pltpu.VMEM((CH,), jnp.int32),
                       pltpu.VMEM((CH,), jnp.int32),
                       pltpu.SemaphoreType.DMA,
                       pltpu.SemaphoreType.DMA],
    )
    def sc_layer(xp_hbm, v_hbm, e_hbm, vs_hbm, es_hbm, s_hbm, ivd_hbm,
                 out_hbm, xp, ze, sbuf, ivdbuf, vb0, vb1, eb0, eb1,
                 sem0, sem1):
        wid = _wid()
        base = wid * fpt * VP
        pltpu.sync_copy(xp_hbm.at[pl.ds(base, fpt * VP)], xp)
        pltpu.sync_copy(s_hbm, sbuf)
        if apply_ivd:
            pltpu.sync_copy(ivd_hbm, ivdbuf)

        def pass_a(vb, eb):
            @plsc.parallel_loop(0, gpc, unroll=4)
            def _(g):
                vv = vb[pl.ds(g * 16, 16)]
                ve = eb[pl.ds(g * 16, 16)]
                for f in range(fpt):
                    vals = plsc.load_gather(xp, [vv + (f * VP)])
                    plsc.addupdate_scatter(ze, [ve + (f * EP)], vals)

        def pass_b(vb, eb):
            @plsc.parallel_loop(0, gpc, unroll=4)
            def _(g):
                vv = vb[pl.ds(g * 16, 16)]
                ve = eb[pl.ds(g * 16, 16)]
                for f in range(fpt):
                    vals = plsc.load_gather(ze, [ve + (f * EP)])
                    plsc.addupdate_scatter(xp, [vv + (f * VP)], vals)

        def run_pass(chunk_fn, vsrc, esrc):
            bufs = ((vb0, eb0, sem0), (vb1, eb1, sem1))
            handles = {}

            def start(c):
                vb, eb, sem = bufs[c & 1]
                h1 = pltpu.async_copy(vsrc.at[pl.ds(c * CH, CH)], vb, sem)
                h2 = pltpu.async_copy(esrc.at[pl.ds(c * CH, CH)], eb, sem)
                handles[c] = (h1, h2)

            start(0)
            for c in range(NCH):
                h1, h2 = handles.pop(c)
                h1.wait()
                h2.wait()
                if c + 1 < NCH:
                    start(c + 1)
                vb, eb, _ = bufs[c & 1]
                chunk_fn(vb, eb)

        _zero_fill(ze, fpt * EP // 16)
        run_pass(pass_a, v_hbm, e_hbm)

        for f in range(fpt):
            @plsc.parallel_loop(0, EP // 16, unroll=4)
            def _(i, f=f):
                off = f * EP + i * 16
                ze[pl.ds(off, 16)] = ze[pl.ds(off, 16)] * sbuf[pl.ds(i * 16, 16)]

        _zero_fill(xp, fpt * VP // 16)
        run_pass(pass_b, vs_hbm, es_hbm)

        if apply_ivd:
            for f in range(fpt):
                @plsc.parallel_loop(0, VP // 16, unroll=4)
                def _(i, f=f):
                    off = f * VP + i * 16
                    xp[pl.ds(off, 16)] = (xp[pl.ds(off, 16)]
                                          * ivdbuf[pl.ds(i * 16, 16)])
        pltpu.sync_copy(xp, out_hbm.at[pl.ds(base, fpt * VP)])

    return sc_layer


_sc_layer_hidden = _make_sc_layer(D_IN, apply_ivd=False)
_sc_layer_final = _make_sc_layer(N_CLS, apply_ivd=True)


def _mm_body(wt_ref, x_ref, b_ref, o_ref):
    o_ref[...] = jnp.dot(wt_ref[...], x_ref[...],
                         preferred_element_type=jnp.float32) + b_ref[...]


def _mm_fused_body(wt_ref, x_ref, b_ref, ivd_ref, o_ref):
    x = jnp.maximum(x_ref[...], 0.0) * ivd_ref[...]
    o_ref[...] = jnp.dot(wt_ref[...], x,
                         preferred_element_type=jnp.float32) + b_ref[...]


def _mm(wt, x2d, b2d, ivd2d=None):
    do, di = wt.shape
    bv = 2560
    if ivd2d is None:
        return pl.pallas_call(
            _mm_body,
            grid=(VP // bv,),
            in_specs=[pl.BlockSpec((do, di), lambda j: (0, 0)),
                      pl.BlockSpec((di, bv), lambda j: (0, j)),
                      pl.BlockSpec((do, 1), lambda j: (0, 0))],
            out_specs=pl.BlockSpec((do, bv), lambda j: (0, j)),
            out_shape=jax.ShapeDtypeStruct((do, VP), jnp.float32),
        )(wt, x2d, b2d)
    return pl.pallas_call(
        _mm_fused_body,
        grid=(VP // bv,),
        in_specs=[pl.BlockSpec((do, di), lambda j: (0, 0)),
                  pl.BlockSpec((di, bv), lambda j: (0, j)),
                  pl.BlockSpec((do, 1), lambda j: (0, 0)),
                  pl.BlockSpec((1, bv), lambda j: (0, j))],
        out_specs=pl.BlockSpec((do, bv), lambda j: (0, j)),
        out_shape=jax.ShapeDtypeStruct((do, VP), jnp.float32),
    )(wt, x2d, b2d, ivd2d)


def _comb1_body(dvp_ref, esp_ref, dv_ref, ivd_ref, ies_ref):
    dv = jnp.maximum(jnp.sum(dvp_ref[...], axis=0, keepdims=True), 1.0)
    dv_ref[...] = dv
    ivd_ref[...] = lax.rsqrt(dv)
    es = jnp.maximum(jnp.sum(esp_ref[...], axis=0, keepdims=True), 1.0)
    ies_ref[...] = 1.0 / es


def _comb1(dvp, esp):
    return pl.pallas_call(
        _comb1_body,
        out_shape=(jax.ShapeDtypeStruct((1, VP), jnp.float32),
                   jax.ShapeDtypeStruct((1, VP), jnp.float32),
                   jax.ShapeDtypeStruct((1, EP), jnp.float32)),
    )(dvp, esp)


def _comb2_body(tep_ref, ies_ref, s_ref):
    te = jnp.sum(tep_ref[...], axis=0, keepdims=True)
    ies = ies_ref[...]
    tilde = jnp.maximum(te * ies, 1.0)
    s_ref[...] = lax.rsqrt(tilde) * ies


def _comb2(tep, ies):
    return pl.pallas_call(
        _comb2_body,
        out_shape=jax.ShapeDtypeStruct((1, EP), jnp.float32),
    )(tep, ies)


def kernel(X, v_idx, e_idx, W0, b0, W1, b1, W2, b2, W3, b3):
    gpc = CH // 16

    def _stride(a):
        return a.reshape(NCH, 16, gpc).transpose(0, 2, 1).reshape(-1)

    v_so = v_idx.astype(jnp.int32)
    e_so = e_idx.astype(jnp.int32)
    v32 = _stride(v_so)
    e32 = _stride(e_so)

    dvp, esp = _sc_hist(v32, e32)
    dv2, ivd2, ies2 = _comb1(dvp.reshape(NW, VP), esp.reshape(NW, EP))
    tep = _sc_te(v32, e32, dv2.reshape(VP))
    s2 = _comb2(tep.reshape(NW, EP), ies2)
    s1 = s2.reshape(EP)
    ivd1 = ivd2.reshape(VP)

    h = jnp.zeros((D_IN, VP), jnp.float32).at[:, :N_V].set(X.T)
    layers = [(W0, b0), (W1, b1), (W2, b2), (W3, b3)]
    for l, (W, b) in enumerate(layers):
        xp = _mm(W.T, h, b[:, None], ivd2d=None if l == 0 else ivd2)
        do = W.shape[1]
        sc = _sc_layer_final if l == 3 else _sc_layer_hidden
        flat = sc(xp.reshape(do * VP), v32, e32, v_so, e_so, s1, ivd1)
        h = flat.reshape(do, VP)
    return h[:, :N_V].T
